# baseline (speedup 1.0000x reference)
import functools
import math

import jax
import jax.numpy as jnp
from jax import lax
from jax.experimental import pallas as pl
from jax.experimental.pallas import tpu as pltpu
from jax.experimental.pallas import tpu_sc as plsc

ROWS = 320000
DIM = 128
NSEG = 10000
NRFF = 64
D1 = 32
D2 = 16
YW = 2 * D2

RBLK = 2000
NBLK = ROWS // RBLK

NC, NS = 2, 16
NW = NC * NS
CHUNK = 128
NCHUNK = ROWS // CHUNK
KMAX = 88
IDXPAD = 2504
NPAD = 10112
TROWS = NPAD // NS


def _dense_body(x_ref, o1_ref, b1_ref, w1_ref, wlv1_ref, o2_ref, b2_ref,
                w2_ref, wlv2_ref, y_ref):
    f32 = jnp.float32
    hi = lax.Precision.HIGHEST
    x = x_ref[...]
    c = math.sqrt(2.0 / NRFF)
    proj1 = jnp.dot(x, o1_ref[...], precision=hi, preferred_element_type=f32)
    proj1 = proj1 + b1_ref[...]
    phi1 = c * jnp.cos(proj1)
    m1 = jnp.dot(phi1, w1_ref[...], precision=hi, preferred_element_type=f32)
    v1 = jnp.dot(phi1 * phi1, jnp.exp(wlv1_ref[...]), precision=hi,
                 preferred_element_type=f32)
    o2 = o2_ref[...]
    proj2 = jnp.dot(m1, o2, precision=hi, preferred_element_type=f32)
    proj2 = proj2 + b2_ref[...]
    pvar2 = jnp.dot(v1, o2 * o2, precision=hi, preferred_element_type=f32)
    phi2 = c * jnp.cos(proj2) * jnp.exp(-0.5 * pvar2)
    m2 = jnp.dot(phi2, w2_ref[...], precision=hi, preferred_element_type=f32)
    v2 = jnp.dot(phi2 * phi2, jnp.exp(wlv2_ref[...]), precision=hi,
                 preferred_element_type=f32)
    norm = jnp.sqrt(jnp.sum(m2 * m2, axis=1, keepdims=True))
    m2 = m2 / jnp.maximum(norm, 1e-12)
    inv = 1.0 / v2
    y_ref[...] = jnp.concatenate([inv * m2, inv], axis=1)


def _dense(x, o1, b1, w1, wlv1, o2, b2, w2, wlv2):
    def wspec(a):
        return pl.BlockSpec(a.shape, lambda i: (0, 0))
    return pl.pallas_call(
        _dense_body,
        grid=(NBLK,),
        in_specs=[
            pl.BlockSpec((RBLK, DIM), lambda i: (i, 0)),
            wspec(o1), wspec(b1), wspec(w1), wspec(wlv1),
            wspec(o2), wspec(b2), wspec(w2), wspec(wlv2),
        ],
        out_specs=pl.BlockSpec((RBLK, YW), lambda i: (i, 0)),
        out_shape=jax.ShapeDtypeStruct((ROWS, YW), jnp.float32),
    )(x, o1, b1, w1, wlv1, o2, b2, w2, wlv2)


def _chunk_lo(w):
    return ((w * (NCHUNK // 4)) >> 6) << 3


def _scatter_body(y_hbm, idx_hbm, zeros_hbm, out_hbm, idx_v, rows_v, acc):
    c = lax.axis_index("c")
    s = lax.axis_index("s")
    wid = s * NC + c
    srow = pl.multiple_of(s * TROWS, 8)
    pltpu.sync_copy(zeros_hbm.at[pl.ds(srow, TROWS)],
                    acc.at[pl.ds(srow, TROWS)])
    b0 = pl.multiple_of(_chunk_lo(wid), 8)
    bn = _chunk_lo(wid + 1)
    k = jnp.where(wid == NW - 1, NCHUNK, bn) - b0
    pltpu.sync_copy(idx_hbm.at[pl.ds(b0, KMAX)], idx_v)
    plsc.subcore_barrier()

    def body(j, carry):
        rbase = pl.multiple_of((b0 + j) * CHUNK, 8)
        pltpu.sync_copy(y_hbm.at[pl.ds(rbase, CHUNK)], rows_v)
        pltpu.sync_copy(rows_v, acc.at[idx_v.at[j]], add=True)
        return carry

    lax.fori_loop(0, k, body, 0)

    plsc.subcore_barrier()
    pltpu.sync_copy(acc.at[pl.ds(srow, TROWS)],
                    out_hbm.at[c].at[pl.ds(srow, TROWS)])


_scatter = functools.partial(
    pl.kernel,
    out_type=jax.ShapeDtypeStruct((NC, NPAD, YW), jnp.float32),
    mesh=plsc.VectorSubcoreMesh(core_axis_name="c", subcore_axis_name="s"),
    scratch_types=[
        pltpu.VMEM((KMAX, CHUNK), jnp.int32),
        pltpu.VMEM((CHUNK, YW), jnp.float32),
        pltpu.VMEM_SHARED((NPAD, YW), jnp.float32),
    ],
)(_scatter_body)


def _finalize_body(p_ref, emb_ref, ev_ref):
    tot = p_ref[0] + p_ref[1]
    wm = tot[:NSEG, :D2]
    si = tot[:NSEG, D2:]
    ev = 1.0 / si
    ev_ref[...] = ev
    emb_ref[...] = ev * wm


def _finalize(part):
    return pl.pallas_call(
        _finalize_body,
        out_shape=(jax.ShapeDtypeStruct((NSEG, D2), jnp.float32),
                   jax.ShapeDtypeStruct((NSEG, D2), jnp.float32)),
    )(part)


def kernel(x, x_idx, Omega1, b1, W1, Wlv1, Omega2, b2, W2, Wlv2):
    y = _dense(x, Omega1, b1.reshape(1, NRFF), W1, Wlv1,
               Omega2, b2.reshape(1, NRFF), W2, Wlv2)
    idx2 = x_idx.astype(jnp.int32).reshape(NCHUNK, CHUNK)
    idx2 = jnp.concatenate(
        [idx2, jnp.zeros((IDXPAD - NCHUNK, CHUNK), jnp.int32)], axis=0)
    zeros = jnp.zeros((NPAD, YW), jnp.float32)
    part = _scatter(y, idx2, zeros)
    emb, ev = _finalize(part)
    return emb, ev
